# baseline (speedup 1.0000x reference)
import functools

import jax
import jax.numpy as jnp
from jax import lax
from jax.experimental import pallas as pl
from jax.experimental.pallas import tpu as pltpu
from jax.experimental.pallas import tpu_sc as plsc

_RADIUS2 = 0.2 * 0.2
_NSAMPLE = 32
_DP = 80
_NC, _NS, _L = 2, 16, 16
_NW = _NC * _NS

@functools.cache
def _sc_mesh():
    return plsc.VectorSubcoreMesh(
        core_axis_name="c", subcore_axis_name="s", num_cores=_NC, num_subcores=_NS
    )


def _take(v, idx):
    return lax.gather(
        v, idx[:, None],
        lax.GatherDimensionNumbers(
            offset_dims=(), collapsed_slice_dims=(0,), start_index_map=(0,)),
        (1,), mode=lax.GatherScatterMode.PROMISE_IN_BOUNDS)


def _byte_tables():
    import numpy as np

    a = np.zeros((256, _L), np.int32)
    for byte in range(256):
        pos = [i for i in range(8) if byte >> i & 1]
        a[byte, : len(pos)] = pos
        a[byte, _L - 1] = len(pos)
    b = a.copy()
    b[:, :8] += 8
    return jnp.asarray(a.reshape(-1)), jnp.asarray(b.reshape(-1))


def _make_ballq2(B, N, interpret=False):
    QW = (B * N) // _NW
    WPB = N // QW
    SCH = 8
    NSC = N // (_L * SCH)
    NP = N + _L

    @functools.partial(
        pl.kernel,
        mesh=_sc_mesh(),
        out_type=jax.ShapeDtypeStruct((B * N * _NSAMPLE,), jnp.int32),
        scratch_types=[
            pltpu.VMEM((NP,), jnp.float32),
            pltpu.VMEM((NP,), jnp.float32),
            pltpu.VMEM((NP,), jnp.float32),
            pltpu.VMEM((256 * _L,), jnp.int32),
            pltpu.VMEM((256 * _L,), jnp.int32),
            pltpu.VMEM((64,), jnp.int32),
            pltpu.VMEM((64,), jnp.int32),
            pltpu.VMEM((QW * _NSAMPLE,), jnp.int32),
            pltpu.SMEM((4,), jnp.int32),
        ],
        interpret=interpret,
    )
    def ballq(xs_hbm, ys_hbm, zs_hbm, ta_hbm, tb_hbm, out_hbm, xs_v, ys_v, zs_v,
              ta_v, tb_v, buf0_v, buf1_v, out_v, sm):
        wid = lax.axis_index("s") * _NC + lax.axis_index("c")
        b = wid // WPB
        qoff = (wid % WPB) * QW
        pltpu.sync_copy(xs_hbm.at[b], xs_v)
        pltpu.sync_copy(ys_hbm.at[b], ys_v)
        pltpu.sync_copy(zs_hbm.at[b], zs_v)
        pltpu.sync_copy(ta_hbm, ta_v)
        pltpu.sync_copy(tb_hbm, tb_v)
        gbase = b * N
        iota = lax.iota(jnp.int32, _L)
        pow2 = jnp.left_shift(jnp.ones((_L,), jnp.int32), iota)
        bfly_idx = [jnp.bitwise_xor(iota, s) for s in (1, 2, 4, 8)]

        def compact(m, vals):
            pkv = jnp.where(m, pow2, 0)
            for bidx in bfly_idx:
                pkv = pkv | _take(pkv, bidx)
            pk = pkv[0]
            row_a = ta_v[pl.ds((pk & 255) * _L, _L)]
            row_b = tb_v[pl.ds((pk >> 8) * _L, _L)]
            clo = row_a[_L - 1]
            perm = jnp.where(
                iota < clo, row_a, _take(row_b, jnp.maximum(iota - clo, 0)))
            return _take(vals, perm), clo + row_b[_L - 1]

        def qbody(qp, carry):
            q0 = qp * 2
            q1 = q0 + 1
            qv0 = [c[pl.ds(qoff + q0, _L)] for c in (xs_v, ys_v, zs_v)]
            qv1 = [c[pl.ds(qoff + q1, _L)] for c in (xs_v, ys_v, zs_v)]
            qx0, qy0, qz0 = (v[0] for v in qv0)
            qx1, qy1, qz1 = (v[0] for v in qv1)
            sm[0] = 0
            sm[1] = 0

            def superchunk(j, c2):
                wp0s = sm[0]
                wp1s = sm[1]

                @pl.when(jnp.minimum(wp0s, wp1s) < _NSAMPLE)
                def _():
                    wp0 = wp0s
                    wp1 = wp1s
                    for u in range(SCH):
                        base = (j * SCH + u) * _L
                        cx = xs_v[pl.ds(base, _L)]
                        cy = ys_v[pl.ds(base, _L)]
                        cz = zs_v[pl.ds(base, _L)]
                        vals = gbase + base + iota
                        dx0 = cx - qx0
                        dy0 = cy - qy0
                        dz0 = cz - qz0
                        dx1 = cx - qx1
                        dy1 = cy - qy1
                        dz1 = cz - qz1
                        d0 = dx0 * dx0 + dy0 * dy0 + dz0 * dz0
                        d1 = dx1 * dx1 + dy1 * dy1 + dz1 * dz1
                        c0, n0 = compact(d0 < _RADIUS2, vals)
                        c1, n1 = compact(d1 < _RADIUS2, vals)
                        buf0_v[pl.ds(jnp.minimum(wp0, _NSAMPLE), _L)] = c0
                        buf1_v[pl.ds(jnp.minimum(wp1, _NSAMPLE), _L)] = c1
                        wp0 = wp0 + n0
                        wp1 = wp1 + n1
                    sm[0] = wp0
                    sm[1] = wp1

                return c2

            lax.fori_loop(0, NSC, superchunk, jnp.int32(0))
            for q, buf, slot in ((q0, buf0_v, 0), (q1, buf1_v, 1)):
                cnt = jnp.minimum(sm[slot], _NSAMPLE)
                v0 = buf[pl.ds(0, _L)]
                v1 = buf[pl.ds(_L, _L)]
                first = v0[0]
                out_v[pl.ds(q * _NSAMPLE, _L)] = jnp.where(iota < cnt, v0, first)
                out_v[pl.ds(q * _NSAMPLE + _L, _L)] = jnp.where(
                    iota + _L < cnt, v1, first)
            return carry

        lax.fori_loop(0, QW // 2, qbody, jnp.int32(0))
        pltpu.sync_copy(out_v, out_hbm.at[pl.ds(wid * QW * _NSAMPLE, QW * _NSAMPLE)])

    return ballq


def _make_gather(P, interpret=False):
    CH = P // _NW
    CK = 128
    NT = CH // CK

    @functools.partial(
        pl.kernel,
        mesh=_sc_mesh(),
        out_type=jax.ShapeDtypeStruct((P, _DP), jnp.float32),
        scratch_types=[
            pltpu.VMEM((CK,), jnp.int32),
            pltpu.VMEM((CK, _DP), jnp.float32),
            pltpu.SemaphoreType.DMA,
        ],
        compiler_params=pltpu.CompilerParams(use_tc_tiling_on_sc=False),
        interpret=interpret,
    )
    def gather(tab_hbm, idx_hbm, out_hbm, idx_v, rows_v, sem):
        wid = lax.axis_index("s") * _NC + lax.axis_index("c")

        def body(t, carry):
            off = wid * CH + t * CK
            pltpu.sync_copy(idx_hbm.at[pl.ds(off, CK)], idx_v)
            pltpu.async_copy(tab_hbm.at[idx_v], rows_v, sem).wait()
            pltpu.sync_copy(rows_v, out_hbm.at[pl.ds(off, CK)])
            return carry

        lax.fori_loop(0, NT, body, jnp.int32(0))

    return gather


def _mlp_body(g_ref, mod_ref, cx_ref, w0_ref, b0_ref, w1_ref, b1_ref, w2_ref,
              b2_ref, out_ref):
    Q = mod_ref.shape[0]
    P = Q * _NSAMPLE
    g = g_ref[...]
    mod = jnp.reshape(
        jnp.broadcast_to(mod_ref[...][:, None, :], (Q, _NSAMPLE, _DP)), (P, _DP))
    cx = jnp.reshape(
        jnp.broadcast_to(cx_ref[...][:, None, :], (Q, _NSAMPLE, _DP)), (P, _DP))
    a = g - mod
    dsq = a * a
    h1 = jnp.maximum(
        jnp.dot(dsq, w0_ref[...], preferred_element_type=jnp.float32)
        + b0_ref[...], 0.0)
    h2 = jnp.maximum(
        jnp.dot(h1, w1_ref[...], preferred_element_type=jnp.float32)
        + b1_ref[...], 0.0)
    h3 = jnp.maximum(
        jnp.sum(h2 * w2_ref[...], axis=1, keepdims=True) + b2_ref[...], 0.0)
    rel = a + cx
    num = jnp.sum(jnp.reshape(rel * h3, (Q, _NSAMPLE, _DP)), axis=1)
    den = jnp.sum(jnp.reshape(h3, (Q, _NSAMPLE)), axis=1, keepdims=True)
    out_ref[...] = num / den


def _make_mlp(BN, Q, interpret=False):
    grid = (BN // Q,)
    return pl.pallas_call(
        _mlp_body,
        grid=grid,
        in_specs=[
            pl.BlockSpec((Q * _NSAMPLE, _DP), lambda i: (i, 0)),
            pl.BlockSpec((Q, _DP), lambda i: (i, 0)),
            pl.BlockSpec((Q, _DP), lambda i: (i, 0)),
            pl.BlockSpec((_DP, 64), lambda i: (0, 0)),
            pl.BlockSpec((1, 64), lambda i: (0, 0)),
            pl.BlockSpec((64, 32), lambda i: (0, 0)),
            pl.BlockSpec((1, 32), lambda i: (0, 0)),
            pl.BlockSpec((1, 32), lambda i: (0, 0)),
            pl.BlockSpec((1, 1), lambda i: (0, 0)),
        ],
        out_specs=pl.BlockSpec((Q, _DP), lambda i: (i, 0)),
        out_shape=jax.ShapeDtypeStruct((BN, _DP), jnp.float32),
        interpret=interpret,
    )


def kernel(xyz, features, W0, b0, W1, b1, W2, b2):
    B, N, _ = xyz.shape
    C = features.shape[1]
    pad = _DP - (C + 3)
    featT = jnp.transpose(features, (0, 2, 1))
    zpad = jnp.zeros((B, N, pad), jnp.float32)
    tab = jnp.concatenate([xyz, featT, zpad], axis=-1).reshape(B * N, _DP)
    mod = jnp.concatenate([2.0 * xyz, featT, zpad], axis=-1).reshape(B * N, _DP)
    cx = jnp.concatenate(
        [xyz, jnp.zeros((B, N, C + pad), jnp.float32)], axis=-1).reshape(B * N, _DP)

    zp = jnp.zeros((B, _L), jnp.float32)
    xs = jnp.concatenate([xyz[..., 0], zp], axis=1)
    ys = jnp.concatenate([xyz[..., 1], zp], axis=1)
    zs = jnp.concatenate([xyz[..., 2], zp], axis=1)
    ta, tb = _byte_tables()
    idxg = _make_ballq2(B, N)(xs, ys, zs, ta, tb)
    g = _make_gather(B * N * _NSAMPLE)(tab, idxg)

    w0p = jnp.pad(W0.T, ((0, pad), (0, 0)))
    outp = _make_mlp(B * N, 256)(
        g, mod, cx, w0p, b0[None, :], W1.T, b1[None, :], W2, b2[None, :])
    return jnp.transpose(outp.reshape(B, N, _DP)[..., :3], (0, 2, 1))


# speedup vs baseline: 1.6038x; 1.6038x over previous
import functools

import jax
import jax.numpy as jnp
from jax import lax
from jax.experimental import pallas as pl
from jax.experimental.pallas import tpu as pltpu
from jax.experimental.pallas import tpu_sc as plsc

_RADIUS2 = 0.2 * 0.2
_NSAMPLE = 32
_DP = 80
_NC, _NS, _L = 2, 16, 16
_NW = _NC * _NS

@functools.cache
def _sc_mesh():
    return plsc.VectorSubcoreMesh(
        core_axis_name="c", subcore_axis_name="s", num_cores=_NC, num_subcores=_NS
    )


def _take(v, idx):
    return lax.gather(
        v, idx[:, None],
        lax.GatherDimensionNumbers(
            offset_dims=(), collapsed_slice_dims=(0,), start_index_map=(0,)),
        (1,), mode=lax.GatherScatterMode.PROMISE_IN_BOUNDS)


def _make_ballq2(B, N, interpret=False):
    QW = (B * N) // _NW
    WPB = N // QW
    SCH = 8
    NSC = N // (_L * SCH)
    NP = N + _L

    @functools.partial(
        pl.kernel,
        mesh=_sc_mesh(),
        out_type=jax.ShapeDtypeStruct((B * N * _NSAMPLE,), jnp.int32),
        scratch_types=[
            pltpu.VMEM((NP,), jnp.float32),
            pltpu.VMEM((NP,), jnp.float32),
            pltpu.VMEM((NP,), jnp.float32),
            pltpu.VMEM((64,), jnp.int32),
            pltpu.VMEM((64,), jnp.int32),
            pltpu.VMEM((QW * _NSAMPLE,), jnp.int32),
            pltpu.SMEM((4,), jnp.int32),
        ],
        interpret=interpret,
    )
    def ballq(xs_hbm, ys_hbm, zs_hbm, out_hbm, xs_v, ys_v, zs_v, buf0_v,
              buf1_v, out_v, sm):
        wid = lax.axis_index("s") * _NC + lax.axis_index("c")
        b = wid // WPB
        qoff = (wid % WPB) * QW
        pltpu.sync_copy(xs_hbm.at[b], xs_v)
        pltpu.sync_copy(ys_hbm.at[b], ys_v)
        pltpu.sync_copy(zs_hbm.at[b], zs_v)
        gbase = b * N
        iota = lax.iota(jnp.int32, _L)
        shift_idx = [jnp.maximum(iota - s, 0) for s in (1, 2, 4, 8)]
        fifteen = jnp.full((_L,), _L - 1, jnp.int32)

        def compact(m, vals):
            cum = jnp.where(m, 1, 0)
            for s, sidx in zip((1, 2, 4, 8), shift_idx):
                cum = cum + jnp.where(iota >= s, _take(cum, sidx), 0)
            pos = jnp.zeros((_L,), jnp.int32)
            for s in (8, 4, 2, 1):
                probe = jnp.minimum(pos + (s - 1), fifteen)
                pos = pos + jnp.where(_take(cum, probe) <= iota, s, 0)
            return _take(vals, pos), cum[_L - 1]

        def qbody(qp, carry):
            q0 = qp * 2
            q1 = q0 + 1
            qv0 = [c[pl.ds(qoff + q0, _L)] for c in (xs_v, ys_v, zs_v)]
            qv1 = [c[pl.ds(qoff + q1, _L)] for c in (xs_v, ys_v, zs_v)]
            qx0, qy0, qz0 = (v[0] for v in qv0)
            qx1, qy1, qz1 = (v[0] for v in qv1)
            sm[0] = 0
            sm[1] = 0

            def superchunk(j, c2):
                wp0s = sm[0]
                wp1s = sm[1]

                @pl.when(jnp.minimum(wp0s, wp1s) < _NSAMPLE)
                def _():
                    wp0 = wp0s
                    wp1 = wp1s
                    for u in range(SCH):
                        base = (j * SCH + u) * _L
                        cx = xs_v[pl.ds(base, _L)]
                        cy = ys_v[pl.ds(base, _L)]
                        cz = zs_v[pl.ds(base, _L)]
                        vals = gbase + base + iota
                        dx0 = cx - qx0
                        dy0 = cy - qy0
                        dz0 = cz - qz0
                        dx1 = cx - qx1
                        dy1 = cy - qy1
                        dz1 = cz - qz1
                        d0 = dx0 * dx0 + dy0 * dy0 + dz0 * dz0
                        d1 = dx1 * dx1 + dy1 * dy1 + dz1 * dz1
                        c0, n0 = compact(d0 < _RADIUS2, vals)
                        c1, n1 = compact(d1 < _RADIUS2, vals)
                        buf0_v[pl.ds(jnp.minimum(wp0, _NSAMPLE), _L)] = c0
                        buf1_v[pl.ds(jnp.minimum(wp1, _NSAMPLE), _L)] = c1
                        wp0 = wp0 + n0
                        wp1 = wp1 + n1
                    sm[0] = wp0
                    sm[1] = wp1

                return c2

            lax.fori_loop(0, NSC, superchunk, jnp.int32(0))
            for q, buf, slot in ((q0, buf0_v, 0), (q1, buf1_v, 1)):
                cnt = jnp.minimum(sm[slot], _NSAMPLE)
                v0 = buf[pl.ds(0, _L)]
                v1 = buf[pl.ds(_L, _L)]
                first = v0[0]
                out_v[pl.ds(q * _NSAMPLE, _L)] = jnp.where(iota < cnt, v0, first)
                out_v[pl.ds(q * _NSAMPLE + _L, _L)] = jnp.where(
                    iota + _L < cnt, v1, first)
            return carry

        lax.fori_loop(0, QW // 2, qbody, jnp.int32(0))
        pltpu.sync_copy(out_v, out_hbm.at[pl.ds(wid * QW * _NSAMPLE, QW * _NSAMPLE)])

    return ballq


def _make_gather(P, interpret=False):
    CH = P // _NW
    CK = 128
    NT = CH // CK

    @functools.partial(
        pl.kernel,
        mesh=_sc_mesh(),
        out_type=jax.ShapeDtypeStruct((P, _DP), jnp.float32),
        scratch_types=[
            pltpu.VMEM((CH,), jnp.int32),
            pltpu.VMEM((2, CK, _DP), jnp.float32),
            pltpu.SemaphoreType.DMA,
            pltpu.SemaphoreType.DMA,
        ],
        compiler_params=pltpu.CompilerParams(use_tc_tiling_on_sc=False),
        interpret=interpret,
    )
    def gather(tab_hbm, idx_hbm, out_hbm, idx_v, rows_v, sem0, sem1):
        wid = lax.axis_index("s") * _NC + lax.axis_index("c")
        woff = wid * CH
        pltpu.sync_copy(idx_hbm.at[pl.ds(woff, CH)], idx_v)
        sems = (sem0, sem1)

        def gather_chunk(t, b):
            return pltpu.make_async_copy(
                tab_hbm.at[idx_v.at[pl.ds(t * CK, CK)]], rows_v.at[b], sems[b])

        gather_chunk(0, 0).start()

        def body(tt, carry):
            for b in (0, 1):
                t = tt * 2 + b

                @pl.when(t + 1 < NT)
                def _():
                    gather_chunk(t + 1, 1 - b).start()

                gather_chunk(t, b).wait()
                pltpu.sync_copy(rows_v.at[b], out_hbm.at[pl.ds(woff + t * CK, CK)])
            return carry

        lax.fori_loop(0, NT // 2, body, jnp.int32(0))

    return gather


def _mlp_body(g_ref, mod_ref, cx_ref, w0_ref, b0_ref, w1_ref, b1_ref, w2_ref,
              b2_ref, out_ref):
    Q = mod_ref.shape[0]
    P = Q * _NSAMPLE
    g = g_ref[...]
    mod = jnp.reshape(
        jnp.broadcast_to(mod_ref[...][:, None, :], (Q, _NSAMPLE, _DP)), (P, _DP))
    a = g - mod
    dsq = a * a
    h1 = jnp.maximum(
        jnp.dot(dsq, w0_ref[...], preferred_element_type=jnp.float32)
        + b0_ref[...], 0.0)
    h2 = jnp.maximum(
        jnp.dot(h1, w1_ref[...], preferred_element_type=jnp.float32)
        + b1_ref[...], 0.0)
    h3 = jnp.maximum(
        jnp.sum(h2 * w2_ref[...], axis=1, keepdims=True) + b2_ref[...], 0.0)
    cx8 = jnp.reshape(
        jnp.broadcast_to(cx_ref[...][:, None, :], (Q, _NSAMPLE, 8)), (P, 8))
    rel8 = lax.slice(a, (0, 0), (P, 8)) + cx8
    lane8 = lax.broadcasted_iota(jnp.int32, (P, 8), 1)
    rel8 = jnp.where(lane8 == 7, 1.0, rel8)
    w8 = rel8 * h3
    num = jnp.sum(jnp.reshape(w8, (Q, _NSAMPLE, 8)), axis=1)
    den = lax.slice(num, (0, 7), (Q, 8))
    out_ref[...] = num / den


def _make_mlp(BN, Q, interpret=False):
    grid = (BN // Q,)
    return pl.pallas_call(
        _mlp_body,
        grid=grid,
        in_specs=[
            pl.BlockSpec((Q * _NSAMPLE, _DP), lambda i: (i, 0)),
            pl.BlockSpec((Q, _DP), lambda i: (i, 0)),
            pl.BlockSpec((Q, 8), lambda i: (i, 0)),
            pl.BlockSpec((_DP, 64), lambda i: (0, 0)),
            pl.BlockSpec((1, 64), lambda i: (0, 0)),
            pl.BlockSpec((64, 32), lambda i: (0, 0)),
            pl.BlockSpec((1, 32), lambda i: (0, 0)),
            pl.BlockSpec((1, 32), lambda i: (0, 0)),
            pl.BlockSpec((1, 1), lambda i: (0, 0)),
        ],
        out_specs=pl.BlockSpec((Q, 8), lambda i: (i, 0)),
        out_shape=jax.ShapeDtypeStruct((BN, 8), jnp.float32),
        interpret=interpret,
    )


def kernel(xyz, features, W0, b0, W1, b1, W2, b2):
    B, N, _ = xyz.shape
    C = features.shape[1]
    pad = _DP - (C + 3)
    featT = jnp.transpose(features, (0, 2, 1))
    zpad = jnp.zeros((B, N, pad), jnp.float32)
    tab = jnp.concatenate([xyz, featT, zpad], axis=-1).reshape(B * N, _DP)
    mod = jnp.concatenate([2.0 * xyz, featT, zpad], axis=-1).reshape(B * N, _DP)
    cx = jnp.concatenate(
        [xyz, jnp.zeros((B, N, 5), jnp.float32)], axis=-1).reshape(B * N, 8)

    zp = jnp.zeros((B, _L), jnp.float32)
    xs = jnp.concatenate([xyz[..., 0], zp], axis=1)
    ys = jnp.concatenate([xyz[..., 1], zp], axis=1)
    zs = jnp.concatenate([xyz[..., 2], zp], axis=1)
    idxg = _make_ballq2(B, N)(xs, ys, zs)
    g = _make_gather(B * N * _NSAMPLE)(tab, idxg)

    w0p = jnp.pad(W0.T, ((0, pad), (0, 0)))
    outp = _make_mlp(B * N, 256)(
        g, mod, cx, w0p, b0[None, :], W1.T, b1[None, :], W2, b2[None, :])
    return jnp.transpose(outp.reshape(B, N, 8)[..., :3], (0, 2, 1))


# speedup vs baseline: 1.6085x; 1.0029x over previous
import functools

import jax
import jax.numpy as jnp
from jax import lax
from jax.experimental import pallas as pl
from jax.experimental.pallas import tpu as pltpu
from jax.experimental.pallas import tpu_sc as plsc

_RADIUS2 = 0.2 * 0.2
_NSAMPLE = 32
_DP = 80
_NC, _NS, _L = 2, 16, 16
_NW = _NC * _NS

@functools.cache
def _sc_mesh():
    return plsc.VectorSubcoreMesh(
        core_axis_name="c", subcore_axis_name="s", num_cores=_NC, num_subcores=_NS
    )


def _take(v, idx):
    return lax.gather(
        v, idx[:, None],
        lax.GatherDimensionNumbers(
            offset_dims=(), collapsed_slice_dims=(0,), start_index_map=(0,)),
        (1,), mode=lax.GatherScatterMode.PROMISE_IN_BOUNDS)


def _make_ballq2(B, N, interpret=False):
    QW = (B * N) // _NW
    WPB = N // QW
    SCH = 8
    NSC = N // (_L * SCH)
    NQ = 4
    NP = N + _L

    @functools.partial(
        pl.kernel,
        mesh=_sc_mesh(),
        out_type=jax.ShapeDtypeStruct((B * N * _NSAMPLE,), jnp.int32),
        scratch_types=[
            pltpu.VMEM((NP,), jnp.float32),
            pltpu.VMEM((NP,), jnp.float32),
            pltpu.VMEM((NP,), jnp.float32),
            pltpu.VMEM((NQ, 64), jnp.int32),
            pltpu.VMEM((QW * _NSAMPLE,), jnp.int32),
            pltpu.SMEM((NQ,), jnp.int32),
        ],
        interpret=interpret,
    )
    def ballq(xs_hbm, ys_hbm, zs_hbm, out_hbm, xs_v, ys_v, zs_v, buf_v, out_v, sm):
        wid = lax.axis_index("s") * _NC + lax.axis_index("c")
        b = wid // WPB
        qoff = (wid % WPB) * QW
        pltpu.sync_copy(xs_hbm.at[b], xs_v)
        pltpu.sync_copy(ys_hbm.at[b], ys_v)
        pltpu.sync_copy(zs_hbm.at[b], zs_v)
        gbase = b * N
        iota = lax.iota(jnp.int32, _L)
        shift_idx = [jnp.maximum(iota - s, 0) for s in (1, 2, 4, 8)]
        fifteen = jnp.full((_L,), _L - 1, jnp.int32)

        def compact(m, vals):
            cum = jnp.where(m, 1, 0)
            for s, sidx in zip((1, 2, 4, 8), shift_idx):
                cum = cum + jnp.where(iota >= s, _take(cum, sidx), 0)
            pos = jnp.zeros((_L,), jnp.int32)
            for s in (8, 4, 2, 1):
                probe = jnp.minimum(pos + (s - 1), fifteen)
                pos = pos + jnp.where(_take(cum, probe) <= iota, s, 0)
            return _take(vals, pos), cum[_L - 1]

        def qbody(qg, carry):
            qs = [qg * NQ + i for i in range(NQ)]
            qc = [[c[pl.ds(qoff + q, _L)][0] for c in (xs_v, ys_v, zs_v)]
                  for q in qs]
            for i in range(NQ):
                sm[i] = 0

            def superchunk(j, c2):
                wps = [sm[i] for i in range(NQ)]
                alive = wps[0]
                for i in range(1, NQ):
                    alive = jnp.minimum(alive, wps[i])

                @pl.when(alive < _NSAMPLE)
                def _():
                    wp = list(wps)
                    for u in range(SCH):
                        base = (j * SCH + u) * _L
                        cx = xs_v[pl.ds(base, _L)]
                        cy = ys_v[pl.ds(base, _L)]
                        cz = zs_v[pl.ds(base, _L)]
                        vals = gbase + base + iota
                        for i in range(NQ):
                            dx = cx - qc[i][0]
                            dy = cy - qc[i][1]
                            dz = cz - qc[i][2]
                            d = dx * dx + dy * dy + dz * dz
                            ci, ni = compact(d < _RADIUS2, vals)
                            buf_v[i, pl.ds(jnp.minimum(wp[i], _NSAMPLE), _L)] = ci
                            wp[i] = wp[i] + ni
                    for i in range(NQ):
                        sm[i] = wp[i]

                return c2

            lax.fori_loop(0, NSC, superchunk, jnp.int32(0))
            for i, q in enumerate(qs):
                cnt = jnp.minimum(sm[i], _NSAMPLE)
                v0 = buf_v[i, pl.ds(0, _L)]
                v1 = buf_v[i, pl.ds(_L, _L)]
                first = v0[0]
                out_v[pl.ds(q * _NSAMPLE, _L)] = jnp.where(iota < cnt, v0, first)
                out_v[pl.ds(q * _NSAMPLE + _L, _L)] = jnp.where(
                    iota + _L < cnt, v1, first)
            return carry

        lax.fori_loop(0, QW // NQ, qbody, jnp.int32(0))
        pltpu.sync_copy(out_v, out_hbm.at[pl.ds(wid * QW * _NSAMPLE, QW * _NSAMPLE)])

    return ballq


def _make_gather(P, interpret=False):
    CH = P // _NW
    CK = 128
    NT = CH // CK

    @functools.partial(
        pl.kernel,
        mesh=_sc_mesh(),
        out_type=jax.ShapeDtypeStruct((P, _DP), jnp.float32),
        scratch_types=[
            pltpu.VMEM((CH,), jnp.int32),
            pltpu.VMEM((2, CK, _DP), jnp.float32),
            pltpu.SemaphoreType.DMA,
            pltpu.SemaphoreType.DMA,
        ],
        compiler_params=pltpu.CompilerParams(use_tc_tiling_on_sc=False),
        interpret=interpret,
    )
    def gather(tab_hbm, idx_hbm, out_hbm, idx_v, rows_v, sem0, sem1):
        wid = lax.axis_index("s") * _NC + lax.axis_index("c")
        woff = wid * CH
        pltpu.sync_copy(idx_hbm.at[pl.ds(woff, CH)], idx_v)
        sems = (sem0, sem1)

        def gather_chunk(t, b):
            return pltpu.make_async_copy(
                tab_hbm.at[idx_v.at[pl.ds(t * CK, CK)]], rows_v.at[b], sems[b])

        gather_chunk(0, 0).start()

        def body(tt, carry):
            for b in (0, 1):
                t = tt * 2 + b

                @pl.when(t + 1 < NT)
                def _():
                    gather_chunk(t + 1, 1 - b).start()

                gather_chunk(t, b).wait()
                pltpu.sync_copy(rows_v.at[b], out_hbm.at[pl.ds(woff + t * CK, CK)])
            return carry

        lax.fori_loop(0, NT // 2, body, jnp.int32(0))

    return gather


def _mlp_body(g_ref, mod_ref, cx_ref, w0_ref, b0_ref, w1_ref, b1_ref, w2_ref,
              b2_ref, out_ref):
    Q = mod_ref.shape[0]
    P = Q * _NSAMPLE
    g = g_ref[...]
    mod = jnp.reshape(
        jnp.broadcast_to(mod_ref[...][:, None, :], (Q, _NSAMPLE, _DP)), (P, _DP))
    a = g - mod
    dsq = a * a
    h1 = jnp.maximum(
        jnp.dot(dsq, w0_ref[...], preferred_element_type=jnp.float32)
        + b0_ref[...], 0.0)
    h2 = jnp.maximum(
        jnp.dot(h1, w1_ref[...], preferred_element_type=jnp.float32)
        + b1_ref[...], 0.0)
    h3 = jnp.maximum(
        jnp.sum(h2 * w2_ref[...], axis=1, keepdims=True) + b2_ref[...], 0.0)
    g8 = lax.slice(g, (0, 0), (P, 8))
    lane8 = lax.broadcasted_iota(jnp.int32, (P, 8), 1)
    w8 = jnp.where(lane8 == 7, h3, g8 * h3)
    num = jnp.sum(jnp.reshape(w8, (Q, _NSAMPLE, 8)), axis=1)
    den = lax.slice(num, (0, 7), (Q, 8))
    out_ref[...] = num / den - cx_ref[...]


def _make_mlp(BN, Q, interpret=False):
    grid = (BN // Q,)
    return pl.pallas_call(
        _mlp_body,
        grid=grid,
        in_specs=[
            pl.BlockSpec((Q * _NSAMPLE, _DP), lambda i: (i, 0)),
            pl.BlockSpec((Q, _DP), lambda i: (i, 0)),
            pl.BlockSpec((Q, 8), lambda i: (i, 0)),
            pl.BlockSpec((_DP, 64), lambda i: (0, 0)),
            pl.BlockSpec((1, 64), lambda i: (0, 0)),
            pl.BlockSpec((64, 32), lambda i: (0, 0)),
            pl.BlockSpec((1, 32), lambda i: (0, 0)),
            pl.BlockSpec((1, 32), lambda i: (0, 0)),
            pl.BlockSpec((1, 1), lambda i: (0, 0)),
        ],
        out_specs=pl.BlockSpec((Q, 8), lambda i: (i, 0)),
        out_shape=jax.ShapeDtypeStruct((BN, 8), jnp.float32),
        interpret=interpret,
    )


def kernel(xyz, features, W0, b0, W1, b1, W2, b2):
    B, N, _ = xyz.shape
    C = features.shape[1]
    pad = _DP - (C + 3)
    featT = jnp.transpose(features, (0, 2, 1))
    zpad = jnp.zeros((B, N, pad), jnp.float32)
    tab = jnp.concatenate([xyz, featT, zpad], axis=-1).reshape(B * N, _DP)
    mod = jnp.concatenate([2.0 * xyz, featT, zpad], axis=-1).reshape(B * N, _DP)
    cx = jnp.concatenate(
        [xyz, jnp.zeros((B, N, 5), jnp.float32)], axis=-1).reshape(B * N, 8)

    zp = jnp.zeros((B, _L), jnp.float32)
    xs = jnp.concatenate([xyz[..., 0], zp], axis=1)
    ys = jnp.concatenate([xyz[..., 1], zp], axis=1)
    zs = jnp.concatenate([xyz[..., 2], zp], axis=1)
    idxg = _make_ballq2(B, N)(xs, ys, zs)
    g = _make_gather(B * N * _NSAMPLE)(tab, idxg)

    w0p = jnp.pad(W0.T, ((0, pad), (0, 0)))
    outp = _make_mlp(B * N, 256)(
        g, mod, cx, w0p, b0[None, :], W1.T, b1[None, :], W2, b2[None, :])
    return jnp.transpose(outp.reshape(B, N, 8)[..., :3], (0, 2, 1))
